# baseline (speedup 1.0000x reference)
import functools

import jax
import jax.numpy as jnp
from jax import lax
from jax.experimental import pallas as pl
from jax.experimental.pallas import tpu as pltpu
from jax.experimental.pallas import tpu_sc as plsc

_NC = 2
_NS = 16
_LANES = 128
_K = 6
_UNROLL = 6
_F = 16
_PK = 8
_DEGW = 16


def _cdiv(a, b):
    return -(-a // b)


@functools.lru_cache(maxsize=None)
def _seg_make(n_pad, rpt_n, feat, cpt0, cpt1):
    f32 = jnp.float32
    mesh = plsc.VectorSubcoreMesh(core_axis_name="c", subcore_axis_name="s",
                                  num_cores=_NC, num_subcores=_NS)
    rpt_e0, rpt_e1 = cpt0 * _K, cpt1 * _K

    def body(hn, src2, dst2, zz, p0, p1, acc,
             src_a, src_b, dst_a, dst_b, dst_c, rows_a, rows_b,
             lsem, gsem, ssem):
        cid = lax.axis_index("c")
        sid = lax.axis_index("s")
        srcs = (src_a, src_b)
        dsts = (dst_a, dst_b, dst_c)
        rows = (rows_a, rows_b)
        nsl = pl.ds(sid * rpt_n, rpt_n)
        pltpu.sync_copy(zz.at[nsl], acc.at[nsl])
        plsc.subcore_barrier()
        row0 = jnp.where(cid == 0, sid * rpt_e0,
                         _NS * rpt_e0 + sid * rpt_e1)
        cpt = jnp.where(cid == 0, cpt0, cpt1)

        def fire_idx(g, b2, b3):
            base = row0 + g * _K
            pltpu.async_copy(src2.at[pl.ds(base, _K)], srcs[b2], lsem)
            pltpu.async_copy(dst2.at[pl.ds(base, _K)], dsts[b3], lsem)

        def drain_scat(b2):
            for j in range(_K):
                pltpu.make_async_copy(zz.at[pl.ds(0, _LANES)],
                                      rows[b2].at[j], ssem).wait()

        def one_chunk(g, u):
            b2, b3 = u % 2, u % 3

            @pl.when(g >= 2)
            def _():
                drain_scat(b2)
            pltpu.make_async_copy(src2.at[pl.ds(0, _K)], srcs[b2], lsem).wait()
            pltpu.make_async_copy(src2.at[pl.ds(0, _K)], dsts[b3], lsem).wait()

            @pl.when(g + 1 < cpt)
            def _():
                fire_idx(g + 1, (u + 1) % 2, (u + 1) % 3)

            gd = [pltpu.async_copy(hn.at[srcs[b2].at[j]], rows[b2].at[j], gsem)
                  for j in range(_K)]
            for j in range(_K):
                gd[j].wait()
            for j in range(_K):
                pltpu.async_copy(rows[b2].at[j], acc.at[dsts[b3].at[j]],
                                 ssem, add=True)

        fire_idx(0, 0, 0)

        def step(gs, carry):
            for u in range(_UNROLL):
                one_chunk(gs * _UNROLL + u, u)
            return carry

        lax.fori_loop(0, cpt // _UNROLL, step, 0)
        for u in range(2):
            drain_scat(u)
        plsc.subcore_barrier()

        @pl.when(cid == 0)
        def _():
            pltpu.sync_copy(acc.at[nsl], p0.at[nsl])

        @pl.when(cid == 1)
        def _():
            pltpu.sync_copy(acc.at[nsl], p1.at[nsl])

    return pl.kernel(
        body,
        out_type=(jax.ShapeDtypeStruct((n_pad, feat), f32),
                  jax.ShapeDtypeStruct((n_pad, feat), f32)),
        mesh=mesh,
        scratch_types=(pltpu.VMEM_SHARED((n_pad, feat), f32),
                       pltpu.VMEM((_K, _LANES), jnp.int32),
                       pltpu.VMEM((_K, _LANES), jnp.int32),
                       pltpu.VMEM((_K, _LANES), jnp.int32),
                       pltpu.VMEM((_K, _LANES), jnp.int32),
                       pltpu.VMEM((_K, _LANES), jnp.int32),
                       pltpu.VMEM((_K, _LANES, feat), f32),
                       pltpu.VMEM((_K, _LANES, feat), f32),
                       pltpu.SemaphoreType.DMA,
                       pltpu.SemaphoreType.DMA,
                       pltpu.SemaphoreType.DMA),
        compiler_params=pltpu.CompilerParams(use_tc_tiling_on_sc=False))


@functools.lru_cache(maxsize=None)
def _deg_make(n_pad, rpt_n, feat, cpt0, cpt1):
    f32 = jnp.float32
    mesh = plsc.VectorSubcoreMesh(core_axis_name="c", subcore_axis_name="s",
                                  num_cores=_NC, num_subcores=_NS)
    rpt_e0, rpt_e1 = cpt0 * _K, cpt1 * _K

    def body(dst2, zz, ones, q0, q1, accd, dst_a, dst_b, dst_c, ones_v,
             lsem, ssem):
        cid = lax.axis_index("c")
        sid = lax.axis_index("s")
        dsts = (dst_a, dst_b, dst_c)
        nsl = pl.ds(sid * rpt_n, rpt_n)
        pltpu.sync_copy(zz.at[nsl], accd.at[nsl])
        pltpu.sync_copy(ones, ones_v)
        plsc.subcore_barrier()
        row0 = jnp.where(cid == 0, sid * rpt_e0,
                         _NS * rpt_e0 + sid * rpt_e1)
        cpt = jnp.where(cid == 0, cpt0, cpt1)

        def fire_idx(g, b3):
            base = row0 + g * _K
            pltpu.async_copy(dst2.at[pl.ds(base, _K)], dsts[b3], lsem)

        def drain_scat():
            for j in range(_K):
                pltpu.make_async_copy(zz.at[pl.ds(0, _LANES)],
                                      ones_v, ssem).wait()

        def one_chunk(g, u):
            b3 = u % 3

            @pl.when(g >= 2)
            def _():
                drain_scat()

            pltpu.make_async_copy(dst2.at[pl.ds(0, _K)], dsts[b3], lsem).wait()

            @pl.when(g + 1 < cpt)
            def _():
                fire_idx(g + 1, (u + 1) % 3)

            for j in range(_K):
                pltpu.async_copy(ones_v, accd.at[dsts[b3].at[j]],
                                 ssem, add=True)

        fire_idx(0, 0)

        def step(gs, carry):
            for u in range(_UNROLL):
                one_chunk(gs * _UNROLL + u, u)
            return carry

        lax.fori_loop(0, cpt // _UNROLL, step, 0)
        for _u in range(2):
            drain_scat()
        plsc.subcore_barrier()

        @pl.when(cid == 0)
        def _():
            pltpu.sync_copy(accd.at[nsl], q0.at[nsl])

        @pl.when(cid == 1)
        def _():
            pltpu.sync_copy(accd.at[nsl], q1.at[nsl])

    return pl.kernel(
        body,
        out_type=(jax.ShapeDtypeStruct((n_pad, feat), f32),
                  jax.ShapeDtypeStruct((n_pad, feat), f32)),
        mesh=mesh,
        scratch_types=(pltpu.VMEM_SHARED((n_pad, feat), f32),
                       pltpu.VMEM((_K, _LANES), jnp.int32),
                       pltpu.VMEM((_K, _LANES), jnp.int32),
                       pltpu.VMEM((_K, _LANES), jnp.int32),
                       pltpu.VMEM((_LANES, feat), f32),
                       pltpu.SemaphoreType.DMA,
                       pltpu.SemaphoreType.DMA),
        compiler_params=pltpu.CompilerParams(use_tc_tiling_on_sc=False))


def _prep_make(deg_w):
    def _prep_body(x_ref, w_ref, q0_ref, q1_ref, hn_ref, d_ref):
        qq = q0_ref[...] + q1_ref[...]
        blk = qq.shape[0]
        dn = lax.rsqrt(qq.reshape(blk, _PK, deg_w)[:, :, :1] + 1.0)
        d = jnp.broadcast_to(dn, (blk, _PK, _F)).reshape(blk, _PK * _F)
        d_ref[...] = d
        hn_ref[...] = jnp.dot(x_ref[...], w_ref[...],
                              preferred_element_type=jnp.float32) * d
    return _prep_body


def _mid_body(p0_ref, p1_ref, hn_ref, d_ref, b_ref, w_ref, o_ref):
    d = d_ref[...]
    t = (p0_ref[...] + p1_ref[...] + hn_ref[...]) * d + b_ref[...]
    t = jnp.maximum(t, 0.0)
    o_ref[...] = jnp.dot(t, w_ref[...], preferred_element_type=jnp.float32) * d


def _fin_make(out_d):
    def _fin_body(p0_ref, p1_ref, hn_ref, d_ref, b_ref, o_ref):
        t = (p0_ref[...] + p1_ref[...] + hn_ref[...]) * d_ref[...] + b_ref[...]
        blk = t.shape[0]
        o_ref[...] = t.reshape(blk, _PK, _F)[:, :, :out_d].reshape(blk * _PK, out_d)
    return _fin_body


def _row_spec(blk):
    return pl.BlockSpec((blk, _PK * _F), lambda i: (i, 0))


def _full_spec(shape):
    return pl.BlockSpec(shape, lambda i: (0, 0))


def _tc_call(body, rows_pk, in_arrays, in_specs, n_out):
    blk = rows_pk // 4
    oshape = jax.ShapeDtypeStruct((rows_pk, _PK * _F), jnp.float32)
    out_shape = [oshape] * n_out if n_out > 1 else oshape
    out_specs = [_row_spec(blk)] * n_out if n_out > 1 else _row_spec(blk)
    return pl.pallas_call(
        body,
        grid=(4,),
        in_specs=in_specs,
        out_specs=out_specs,
        out_shape=out_shape)(*in_arrays)


def kernel(x, edge_index, batch_index, W1, b1, W2, b2, W3, b3):
    f32 = jnp.float32
    n, seq = x.shape
    e = edge_index.shape[1]
    emb = W1.shape[1]
    out_d = W3.shape[1]

    n_pad = _cdiv(n + 1, 1024) * 1024
    rpt_n = n_pad // _NS
    rows_pk = n_pad // _PK
    ct = _cdiv(_cdiv(e, _NS * _K * _LANES), 2 * _UNROLL) * 2 * _UNROLL
    seg_c0a = int(round(ct * 0.74 / _UNROLL)) * _UNROLL
    seg_c0 = int(round(ct * 0.72 / _UNROLL)) * _UNROLL
    deg_c0 = int(round(ct * 0.60 / _UNROLL)) * _UNROLL
    rows2d = _NS * _K * ct
    pad = rows2d * _LANES - e

    src2 = jnp.concatenate(
        [edge_index[0], jnp.zeros((pad,), jnp.int32)]).reshape(rows2d, _LANES)
    dst2 = jnp.concatenate(
        [edge_index[1], jnp.full((pad,), n, jnp.int32)]).reshape(rows2d, _LANES)

    eye8 = jnp.eye(_PK, dtype=f32)
    xp = jnp.pad(x, ((0, n_pad - n), (0, _F - seq))).reshape(rows_pk, _PK * _F)
    W1b = jnp.kron(eye8, jnp.pad(W1, ((0, _F - seq), (0, _F - emb))))
    W2b = jnp.kron(eye8, jnp.pad(W2, ((0, _F - emb), (0, _F - emb))))
    W3b = jnp.kron(eye8, jnp.pad(W3, ((0, _F - emb), (0, _F - out_d))))
    b1b = jnp.tile(jnp.pad(b1, (0, _F - emb)), _PK).reshape(1, _PK * _F)
    b2b = jnp.tile(jnp.pad(b2, (0, _F - emb)), _PK).reshape(1, _PK * _F)
    b3b = jnp.tile(jnp.pad(b3, (0, _F - out_d)), _PK).reshape(1, _PK * _F)

    zz = jnp.zeros((n_pad, _F), f32)
    zd = jnp.zeros((n_pad, _DEGW), f32)
    ones = jnp.ones((_LANES, _DEGW), f32)

    deg_fn = _deg_make(n_pad, rpt_n, _DEGW, deg_c0, ct - deg_c0)
    seg_fn1 = _seg_make(n_pad, rpt_n, _F, seg_c0a, ct - seg_c0a)
    seg_fn = _seg_make(n_pad, rpt_n, _F, seg_c0, ct - seg_c0)

    def pk(a):
        return a.reshape(rows_pk, _PK * _F)

    def unpk(a):
        return a.reshape(n_pad, _F)

    dq0, dq1 = deg_fn(dst2, zd, ones)

    blkq = rows_pk // 4
    qspec = pl.BlockSpec((blkq, _PK * _DEGW), lambda i: (i, 0))
    hn1, dpk = _tc_call(_prep_make(_DEGW), rows_pk,
                        (xp, W1b,
                         dq0.reshape(rows_pk, _PK * _DEGW),
                         dq1.reshape(rows_pk, _PK * _DEGW)),
                        [_row_spec(blkq), _full_spec((_PK * _F, _PK * _F)),
                         qspec, qspec], 2)

    s0, s1 = seg_fn1(unpk(hn1), src2, dst2, zz)
    hn2 = _tc_call(_mid_body, rows_pk, (pk(s0), pk(s1), hn1, dpk, b1b, W2b),
                   [_row_spec(rows_pk // 4)] * 4 +
                   [_full_spec((1, _PK * _F)), _full_spec((_PK * _F, _PK * _F))], 1)

    s0, s1 = seg_fn(unpk(hn2), src2, dst2, zz)
    hn3 = _tc_call(_mid_body, rows_pk, (pk(s0), pk(s1), hn2, dpk, b2b, W3b),
                   [_row_spec(rows_pk // 4)] * 4 +
                   [_full_spec((1, _PK * _F)), _full_spec((_PK * _F, _PK * _F))], 1)

    s0, s1 = seg_fn(unpk(hn3), src2, dst2, zz)
    blk = rows_pk // 4
    outp = pl.pallas_call(
        _fin_make(out_d),
        grid=(4,),
        in_specs=[_row_spec(blk)] * 4 + [_full_spec((1, _PK * _F))],
        out_specs=pl.BlockSpec((blk * _PK, out_d), lambda i: (i, 0)),
        out_shape=jax.ShapeDtypeStruct((n, out_d), jnp.float32),
    )(pk(s0), pk(s1), hn3, dpk, b3b)

    return outp


# speedup vs baseline: 1.0206x; 1.0206x over previous
import functools

import jax
import jax.numpy as jnp
from jax import lax
from jax.experimental import pallas as pl
from jax.experimental.pallas import tpu as pltpu
from jax.experimental.pallas import tpu_sc as plsc

_NC = 2
_NS = 16
_LANES = 128
_K = 6
_UNROLL = 6
_F = 16
_PK = 8
_DEGW = 16


def _cdiv(a, b):
    return -(-a // b)


@functools.lru_cache(maxsize=None)
def _seg_make(n_pad, rpt_n, feat, cpt0, cpt1):
    f32 = jnp.float32
    mesh = plsc.VectorSubcoreMesh(core_axis_name="c", subcore_axis_name="s",
                                  num_cores=_NC, num_subcores=_NS)
    rpt_e0, rpt_e1 = cpt0 * _K, cpt1 * _K

    def body(hn, src2, dst2, zz, p0, p1, acc,
             src_a, src_b, dst_a, dst_b, dst_c, rows_a, rows_b,
             lsem, gsem, ssem):
        cid = lax.axis_index("c")
        sid = lax.axis_index("s")
        srcs = (src_a, src_b)
        dsts = (dst_a, dst_b, dst_c)
        rows = (rows_a, rows_b)
        nsl = pl.ds(sid * rpt_n, rpt_n)
        pltpu.sync_copy(zz.at[nsl], acc.at[nsl])
        plsc.subcore_barrier()
        row0 = jnp.where(cid == 0, sid * rpt_e0,
                         _NS * rpt_e0 + sid * rpt_e1)
        cpt = jnp.where(cid == 0, cpt0, cpt1)

        def fire_idx(g, b2, b3):
            base = row0 + g * _K
            pltpu.async_copy(src2.at[pl.ds(base, _K)], srcs[b2], lsem)
            pltpu.async_copy(dst2.at[pl.ds(base, _K)], dsts[b3], lsem)

        def drain_scat(b2):
            for j in range(_K):
                pltpu.make_async_copy(zz.at[pl.ds(0, _LANES)],
                                      rows[b2].at[j], ssem).wait()

        def one_chunk(g, u):
            b2, b3 = u % 2, u % 3

            @pl.when(g >= 2)
            def _():
                drain_scat(b2)
            pltpu.make_async_copy(src2.at[pl.ds(0, _K)], srcs[b2], lsem).wait()
            pltpu.make_async_copy(src2.at[pl.ds(0, _K)], dsts[b3], lsem).wait()

            @pl.when(g + 1 < cpt)
            def _():
                fire_idx(g + 1, (u + 1) % 2, (u + 1) % 3)

            gd = [pltpu.async_copy(hn.at[srcs[b2].at[j]], rows[b2].at[j], gsem)
                  for j in range(_K)]
            for j in range(_K):
                gd[j].wait()
            for j in range(_K):
                pltpu.async_copy(rows[b2].at[j], acc.at[dsts[b3].at[j]],
                                 ssem, add=True)

        fire_idx(0, 0, 0)

        def step(gs, carry):
            for u in range(_UNROLL):
                one_chunk(gs * _UNROLL + u, u)
            return carry

        lax.fori_loop(0, cpt // _UNROLL, step, 0)
        for u in range(2):
            drain_scat(u)
        plsc.subcore_barrier()

        @pl.when(cid == 0)
        def _():
            pltpu.sync_copy(acc.at[nsl], p0.at[nsl])

        @pl.when(cid == 1)
        def _():
            pltpu.sync_copy(acc.at[nsl], p1.at[nsl])

    return pl.kernel(
        body,
        out_type=(jax.ShapeDtypeStruct((n_pad, feat), f32),
                  jax.ShapeDtypeStruct((n_pad, feat), f32)),
        mesh=mesh,
        scratch_types=(pltpu.VMEM_SHARED((n_pad, feat), f32),
                       pltpu.VMEM((_K, _LANES), jnp.int32),
                       pltpu.VMEM((_K, _LANES), jnp.int32),
                       pltpu.VMEM((_K, _LANES), jnp.int32),
                       pltpu.VMEM((_K, _LANES), jnp.int32),
                       pltpu.VMEM((_K, _LANES), jnp.int32),
                       pltpu.VMEM((_K, _LANES, feat), f32),
                       pltpu.VMEM((_K, _LANES, feat), f32),
                       pltpu.SemaphoreType.DMA,
                       pltpu.SemaphoreType.DMA,
                       pltpu.SemaphoreType.DMA),
        compiler_params=pltpu.CompilerParams(use_tc_tiling_on_sc=False))


@functools.lru_cache(maxsize=None)
def _deg_make(n_pad, rpt_n, feat, cpt0, cpt1):
    f32 = jnp.float32
    mesh = plsc.VectorSubcoreMesh(core_axis_name="c", subcore_axis_name="s",
                                  num_cores=_NC, num_subcores=_NS)
    rpt_e0, rpt_e1 = cpt0 * _K, cpt1 * _K

    def body(dst2, zz, ones, q0, q1, accd, dst_a, dst_b, dst_c, ones_v,
             lsem, ssem):
        cid = lax.axis_index("c")
        sid = lax.axis_index("s")
        dsts = (dst_a, dst_b, dst_c)
        nsl = pl.ds(sid * rpt_n, rpt_n)
        pltpu.sync_copy(zz.at[nsl], accd.at[nsl])
        pltpu.sync_copy(ones, ones_v)
        plsc.subcore_barrier()
        row0 = jnp.where(cid == 0, sid * rpt_e0,
                         _NS * rpt_e0 + sid * rpt_e1)
        cpt = jnp.where(cid == 0, cpt0, cpt1)

        def fire_idx(g, b3):
            base = row0 + g * _K
            pltpu.async_copy(dst2.at[pl.ds(base, _K)], dsts[b3], lsem)

        def drain_scat():
            for j in range(_K):
                pltpu.make_async_copy(zz.at[pl.ds(0, _LANES)],
                                      ones_v, ssem).wait()

        def one_chunk(g, u):
            b3 = u % 3

            @pl.when(g >= 2)
            def _():
                drain_scat()

            pltpu.make_async_copy(dst2.at[pl.ds(0, _K)], dsts[b3], lsem).wait()

            @pl.when(g + 1 < cpt)
            def _():
                fire_idx(g + 1, (u + 1) % 3)

            for j in range(_K):
                pltpu.async_copy(ones_v, accd.at[dsts[b3].at[j]],
                                 ssem, add=True)

        fire_idx(0, 0)

        def step(gs, carry):
            for u in range(_UNROLL):
                one_chunk(gs * _UNROLL + u, u)
            return carry

        lax.fori_loop(0, cpt // _UNROLL, step, 0)
        for _u in range(2):
            drain_scat()
        plsc.subcore_barrier()

        @pl.when(cid == 0)
        def _():
            pltpu.sync_copy(accd.at[nsl], q0.at[nsl])

        @pl.when(cid == 1)
        def _():
            pltpu.sync_copy(accd.at[nsl], q1.at[nsl])

    return pl.kernel(
        body,
        out_type=(jax.ShapeDtypeStruct((n_pad, feat), f32),
                  jax.ShapeDtypeStruct((n_pad, feat), f32)),
        mesh=mesh,
        scratch_types=(pltpu.VMEM_SHARED((n_pad, feat), f32),
                       pltpu.VMEM((_K, _LANES), jnp.int32),
                       pltpu.VMEM((_K, _LANES), jnp.int32),
                       pltpu.VMEM((_K, _LANES), jnp.int32),
                       pltpu.VMEM((_LANES, feat), f32),
                       pltpu.SemaphoreType.DMA,
                       pltpu.SemaphoreType.DMA),
        compiler_params=pltpu.CompilerParams(use_tc_tiling_on_sc=False))


def _prep_make(deg_w):
    del deg_w

    def _prep_body(x_ref, w_ref, q0_ref, q1_ref, hn_ref, d_ref):
        d = lax.rsqrt(q0_ref[...] + q1_ref[...] + 1.0)
        d_ref[...] = d
        hn_ref[...] = jnp.dot(x_ref[...], w_ref[...],
                              preferred_element_type=jnp.float32) * d
    return _prep_body


def _mid_body(p0_ref, p1_ref, hn_ref, d_ref, b_ref, w_ref, o_ref):
    d = d_ref[...]
    t = (p0_ref[...] + p1_ref[...] + hn_ref[...]) * d + b_ref[...]
    t = jnp.maximum(t, 0.0)
    o_ref[...] = jnp.dot(t, w_ref[...], preferred_element_type=jnp.float32) * d


def _fin_make(out_d):
    def _fin_body(p0_ref, p1_ref, hn_ref, d_ref, b_ref, o_ref):
        t = (p0_ref[...] + p1_ref[...] + hn_ref[...]) * d_ref[...] + b_ref[...]
        blk = t.shape[0]
        o_ref[...] = t.reshape(blk, _PK, _F)[:, :, :out_d].reshape(blk * _PK, out_d)
    return _fin_body


def _row_spec(blk):
    return pl.BlockSpec((blk, _PK * _F), lambda i: (i, 0))


def _full_spec(shape):
    return pl.BlockSpec(shape, lambda i: (0, 0))


def _tc_call(body, rows_pk, in_arrays, in_specs, n_out):
    blk = rows_pk // 4
    oshape = jax.ShapeDtypeStruct((rows_pk, _PK * _F), jnp.float32)
    out_shape = [oshape] * n_out if n_out > 1 else oshape
    out_specs = [_row_spec(blk)] * n_out if n_out > 1 else _row_spec(blk)
    return pl.pallas_call(
        body,
        grid=(4,),
        in_specs=in_specs,
        out_specs=out_specs,
        out_shape=out_shape)(*in_arrays)


def kernel(x, edge_index, batch_index, W1, b1, W2, b2, W3, b3):
    f32 = jnp.float32
    n, seq = x.shape
    e = edge_index.shape[1]
    emb = W1.shape[1]
    out_d = W3.shape[1]

    n_pad = _cdiv(n + 1, 1024) * 1024
    rpt_n = n_pad // _NS
    rows_pk = n_pad // _PK
    ct = _cdiv(_cdiv(e, _NS * _K * _LANES), 2 * _UNROLL) * 2 * _UNROLL
    seg_c0 = int(round(ct * 0.72 / _UNROLL)) * _UNROLL
    deg_c0 = int(round(ct * 0.60 / _UNROLL)) * _UNROLL
    rows2d = _NS * _K * ct
    pad = rows2d * _LANES - e

    src2 = jnp.concatenate(
        [edge_index[0], jnp.zeros((pad,), jnp.int32)]).reshape(rows2d, _LANES)
    dst2 = jnp.concatenate(
        [edge_index[1], jnp.full((pad,), n, jnp.int32)]).reshape(rows2d, _LANES)

    eye8 = jnp.eye(_PK, dtype=f32)
    xp = jnp.pad(x, ((0, n_pad - n), (0, _F - seq))).reshape(rows_pk, _PK * _F)
    W1b = jnp.kron(eye8, jnp.pad(W1, ((0, _F - seq), (0, _F - emb))))
    W2b = jnp.kron(eye8, jnp.pad(W2, ((0, _F - emb), (0, _F - emb))))
    W3b = jnp.kron(eye8, jnp.pad(W3, ((0, _F - emb), (0, _F - out_d))))
    b1b = jnp.tile(jnp.pad(b1, (0, _F - emb)), _PK).reshape(1, _PK * _F)
    b2b = jnp.tile(jnp.pad(b2, (0, _F - emb)), _PK).reshape(1, _PK * _F)
    b3b = jnp.tile(jnp.pad(b3, (0, _F - out_d)), _PK).reshape(1, _PK * _F)

    zz = jnp.zeros((n_pad, _F), f32)
    zd = jnp.zeros((n_pad, _DEGW), f32)
    ones = jnp.ones((_LANES, _DEGW), f32)

    deg_fn = _deg_make(n_pad, rpt_n, _DEGW, deg_c0, ct - deg_c0)
    seg_fn = _seg_make(n_pad, rpt_n, _F, seg_c0, ct - seg_c0)
    seg_fn1 = seg_fn

    def pk(a):
        return a.reshape(rows_pk, _PK * _F)

    def unpk(a):
        return a.reshape(n_pad, _F)

    dq0, dq1 = deg_fn(dst2, zd, ones)

    blkq = rows_pk // 4
    qspec = pl.BlockSpec((blkq, _PK * _DEGW), lambda i: (i, 0))
    hn1, dpk = _tc_call(_prep_make(_DEGW), rows_pk,
                        (xp, W1b,
                         dq0.reshape(rows_pk, _PK * _DEGW),
                         dq1.reshape(rows_pk, _PK * _DEGW)),
                        [_row_spec(blkq), _full_spec((_PK * _F, _PK * _F)),
                         qspec, qspec], 2)

    s0, s1 = seg_fn1(unpk(hn1), src2, dst2, zz)
    hn2 = _tc_call(_mid_body, rows_pk, (pk(s0), pk(s1), hn1, dpk, b1b, W2b),
                   [_row_spec(rows_pk // 4)] * 4 +
                   [_full_spec((1, _PK * _F)), _full_spec((_PK * _F, _PK * _F))], 1)

    s0, s1 = seg_fn(unpk(hn2), src2, dst2, zz)
    hn3 = _tc_call(_mid_body, rows_pk, (pk(s0), pk(s1), hn2, dpk, b2b, W3b),
                   [_row_spec(rows_pk // 4)] * 4 +
                   [_full_spec((1, _PK * _F)), _full_spec((_PK * _F, _PK * _F))], 1)

    s0, s1 = seg_fn(unpk(hn3), src2, dst2, zz)
    blk = rows_pk // 4
    outp = pl.pallas_call(
        _fin_make(out_d),
        grid=(4,),
        in_specs=[_row_spec(blk)] * 4 + [_full_spec((1, _PK * _F))],
        out_specs=pl.BlockSpec((blk * _PK, out_d), lambda i: (i, 0)),
        out_shape=jax.ShapeDtypeStruct((n, out_d), jnp.float32),
    )(pk(s0), pk(s1), hn3, dpk, b3b)

    return outp


# speedup vs baseline: 1.0277x; 1.0070x over previous
import functools

import jax
import jax.numpy as jnp
from jax import lax
from jax.experimental import pallas as pl
from jax.experimental.pallas import tpu as pltpu
from jax.experimental.pallas import tpu_sc as plsc

_NC = 2
_NS = 16
_LANES = 128
_K = 6
_UNROLL = 6
_F = 16
_PK = 8
_DEGW = 16


def _cdiv(a, b):
    return -(-a // b)


@functools.lru_cache(maxsize=None)
def _seg_make(n_pad, rpt_n, feat, cpt0, cpt1):
    f32 = jnp.float32
    mesh = plsc.VectorSubcoreMesh(core_axis_name="c", subcore_axis_name="s",
                                  num_cores=_NC, num_subcores=_NS)
    rpt_e0, rpt_e1 = cpt0 * _K, cpt1 * _K

    def body(hn, src2, dst2, zz, p0, p1, acc,
             src_a, src_b, dst_a, dst_b, dst_c, rows_a, rows_b,
             lsem, gsem, ssem):
        cid = lax.axis_index("c")
        sid = lax.axis_index("s")
        srcs = (src_a, src_b)
        dsts = (dst_a, dst_b, dst_c)
        rows = (rows_a, rows_b)
        nsl = pl.ds(sid * rpt_n, rpt_n)
        pltpu.sync_copy(zz.at[nsl], acc.at[nsl])
        plsc.subcore_barrier()
        row0 = jnp.where(cid == 0, sid * rpt_e0,
                         _NS * rpt_e0 + sid * rpt_e1)
        cpt = jnp.where(cid == 0, cpt0, cpt1)

        def fire_idx(g, b2, b3):
            base = row0 + g * _K
            pltpu.async_copy(src2.at[pl.ds(base, _K)], srcs[b2], lsem)
            pltpu.async_copy(dst2.at[pl.ds(base, _K)], dsts[b3], lsem)

        def drain_scat(b2):
            for j in range(_K):
                pltpu.make_async_copy(zz.at[pl.ds(0, _LANES)],
                                      rows[b2].at[j], ssem).wait()

        def one_chunk(g, u):
            b2, b3 = u % 2, u % 3

            @pl.when(g >= 2)
            def _():
                drain_scat(b2)
            pltpu.make_async_copy(src2.at[pl.ds(0, _K)], srcs[b2], lsem).wait()
            pltpu.make_async_copy(src2.at[pl.ds(0, _K)], dsts[b3], lsem).wait()

            @pl.when(g + 1 < cpt)
            def _():
                fire_idx(g + 1, (u + 1) % 2, (u + 1) % 3)

            gd = [pltpu.async_copy(hn.at[srcs[b2].at[j]], rows[b2].at[j], gsem)
                  for j in range(_K)]
            for j in range(_K):
                gd[j].wait()
            for j in range(_K):
                pltpu.async_copy(rows[b2].at[j], acc.at[dsts[b3].at[j]],
                                 ssem, add=True)

        fire_idx(0, 0, 0)

        def step(gs, carry):
            for u in range(_UNROLL):
                one_chunk(gs * _UNROLL + u, u)
            return carry

        lax.fori_loop(0, cpt // _UNROLL, step, 0)
        for u in range(2):
            drain_scat(u)
        plsc.subcore_barrier()

        @pl.when(cid == 0)
        def _():
            pltpu.sync_copy(acc.at[nsl], p0.at[nsl])

        @pl.when(cid == 1)
        def _():
            pltpu.sync_copy(acc.at[nsl], p1.at[nsl])

    return pl.kernel(
        body,
        out_type=(jax.ShapeDtypeStruct((n_pad, feat), f32),
                  jax.ShapeDtypeStruct((n_pad, feat), f32)),
        mesh=mesh,
        scratch_types=(pltpu.VMEM_SHARED((n_pad, feat), f32),
                       pltpu.VMEM((_K, _LANES), jnp.int32),
                       pltpu.VMEM((_K, _LANES), jnp.int32),
                       pltpu.VMEM((_K, _LANES), jnp.int32),
                       pltpu.VMEM((_K, _LANES), jnp.int32),
                       pltpu.VMEM((_K, _LANES), jnp.int32),
                       pltpu.VMEM((_K, _LANES, feat), f32),
                       pltpu.VMEM((_K, _LANES, feat), f32),
                       pltpu.SemaphoreType.DMA,
                       pltpu.SemaphoreType.DMA,
                       pltpu.SemaphoreType.DMA),
        compiler_params=pltpu.CompilerParams(use_tc_tiling_on_sc=False))


@functools.lru_cache(maxsize=None)
def _deg2_make(n_pad, cpt0, cpt1):
    f32 = jnp.float32
    i32 = jnp.int32
    mesh = plsc.VectorSubcoreMesh(core_axis_name="c", subcore_axis_name="s",
                                  num_cores=_NC, num_subcores=_NS)
    rpt_e0, rpt_e1 = cpt0 * _K, cpt1 * _K
    nr = n_pad // 16
    nrj = nr // _LANES
    rpt_a = nr // _NS
    epc = _K * _LANES

    def body(dst1d, zz, rowidx, q0, q1, acc, hist, dva, dvb, idx_v,
             lsem, ssem):
        cid = lax.axis_index("c")
        sid = lax.axis_index("s")
        dvs = (dva, dvb)
        nsl = pl.ds(sid * rpt_a, rpt_a)
        pltpu.sync_copy(zz.at[nsl], acc.at[nsl])
        pltpu.sync_copy(zz.at[pl.ds(0, nr)], hist)
        pltpu.sync_copy(rowidx, idx_v)
        plsc.subcore_barrier()
        row0 = jnp.where(cid == 0, sid * rpt_e0,
                         _NS * rpt_e0 + sid * rpt_e1)
        cpt = jnp.where(cid == 0, cpt0, cpt1)
        ones16 = jnp.full((16,), 1.0, f32)

        def fire_idx(g, b2):
            pltpu.async_copy(dst1d.at[pl.ds((row0 + g * _K) * _LANES, epc)],
                             dvs[b2], lsem)

        def one_chunk(g, u):
            b2 = u % 2
            pltpu.make_async_copy(dst1d.at[pl.ds(0, epc)], dvs[b2],
                                  lsem).wait()

            @pl.when(g + 1 < cpt)
            def _():
                fire_idx(g + 1, (u + 1) % 2)

            for r in range(epc // 16):
                iv = dvs[b2][pl.ds(r * 16, 16)]
                plsc.addupdate_scatter(
                    hist, (lax.shift_right_logical(iv, 4),
                           lax.bitwise_and(iv, 15)), ones16)

        fire_idx(0, 0)

        def step(gs, carry):
            for u in range(_UNROLL):
                one_chunk(gs * _UNROLL + u, u)
            return carry

        lax.fori_loop(0, cpt // _UNROLL, step, 0)
        rd = [pltpu.async_copy(hist.at[pl.ds(j * _LANES, _LANES)],
                               acc.at[idx_v.at[j]], ssem, add=True)
              for j in range(nrj)]
        for d in rd:
            d.wait()
        plsc.subcore_barrier()

        @pl.when(cid == 0)
        def _():
            pltpu.sync_copy(acc.at[nsl], q0.at[nsl])

        @pl.when(cid == 1)
        def _():
            pltpu.sync_copy(acc.at[nsl], q1.at[nsl])

    return pl.kernel(
        body,
        out_type=(jax.ShapeDtypeStruct((nr, 16), f32),
                  jax.ShapeDtypeStruct((nr, 16), f32)),
        mesh=mesh,
        scratch_types=(pltpu.VMEM_SHARED((nr, 16), f32),
                       pltpu.VMEM((nr, 16), f32),
                       pltpu.VMEM((epc,), i32),
                       pltpu.VMEM((epc,), i32),
                       pltpu.VMEM((nrj, _LANES), i32),
                       pltpu.SemaphoreType.DMA,
                       pltpu.SemaphoreType.DMA),
        compiler_params=pltpu.CompilerParams(use_tc_tiling_on_sc=False,
                                             needs_layout_passes=False))


@functools.lru_cache(maxsize=None)
def _deg_make(n_pad, rpt_n, feat, cpt0, cpt1):
    f32 = jnp.float32
    mesh = plsc.VectorSubcoreMesh(core_axis_name="c", subcore_axis_name="s",
                                  num_cores=_NC, num_subcores=_NS)
    rpt_e0, rpt_e1 = cpt0 * _K, cpt1 * _K

    def body(dst2, zz, ones, q0, q1, accd, dst_a, dst_b, dst_c, ones_v,
             lsem, ssem):
        cid = lax.axis_index("c")
        sid = lax.axis_index("s")
        dsts = (dst_a, dst_b, dst_c)
        nsl = pl.ds(sid * rpt_n, rpt_n)
        pltpu.sync_copy(zz.at[nsl], accd.at[nsl])
        pltpu.sync_copy(ones, ones_v)
        plsc.subcore_barrier()
        row0 = jnp.where(cid == 0, sid * rpt_e0,
                         _NS * rpt_e0 + sid * rpt_e1)
        cpt = jnp.where(cid == 0, cpt0, cpt1)

        def fire_idx(g, b3):
            base = row0 + g * _K
            pltpu.async_copy(dst2.at[pl.ds(base, _K)], dsts[b3], lsem)

        def drain_scat():
            for j in range(_K):
                pltpu.make_async_copy(zz.at[pl.ds(0, _LANES)],
                                      ones_v, ssem).wait()

        def one_chunk(g, u):
            b3 = u % 3

            @pl.when(g >= 2)
            def _():
                drain_scat()

            pltpu.make_async_copy(dst2.at[pl.ds(0, _K)], dsts[b3], lsem).wait()

            @pl.when(g + 1 < cpt)
            def _():
                fire_idx(g + 1, (u + 1) % 3)

            for j in range(_K):
                pltpu.async_copy(ones_v, accd.at[dsts[b3].at[j]],
                                 ssem, add=True)

        fire_idx(0, 0)

        def step(gs, carry):
            for u in range(_UNROLL):
                one_chunk(gs * _UNROLL + u, u)
            return carry

        lax.fori_loop(0, cpt // _UNROLL, step, 0)
        for _u in range(2):
            drain_scat()
        plsc.subcore_barrier()

        @pl.when(cid == 0)
        def _():
            pltpu.sync_copy(accd.at[nsl], q0.at[nsl])

        @pl.when(cid == 1)
        def _():
            pltpu.sync_copy(accd.at[nsl], q1.at[nsl])

    return pl.kernel(
        body,
        out_type=(jax.ShapeDtypeStruct((n_pad, feat), f32),
                  jax.ShapeDtypeStruct((n_pad, feat), f32)),
        mesh=mesh,
        scratch_types=(pltpu.VMEM_SHARED((n_pad, feat), f32),
                       pltpu.VMEM((_K, _LANES), jnp.int32),
                       pltpu.VMEM((_K, _LANES), jnp.int32),
                       pltpu.VMEM((_K, _LANES), jnp.int32),
                       pltpu.VMEM((_LANES, feat), f32),
                       pltpu.SemaphoreType.DMA,
                       pltpu.SemaphoreType.DMA),
        compiler_params=pltpu.CompilerParams(use_tc_tiling_on_sc=False))


def _prep_body(x_ref, w_ref, kr_ref, q0_ref, q1_ref, hn_ref, d_ref):
    dn = lax.rsqrt(q0_ref[...] + q1_ref[...] + 1.0)
    d = jnp.dot(dn, kr_ref[...], preferred_element_type=jnp.float32)
    d_ref[...] = d
    hn_ref[...] = jnp.dot(x_ref[...], w_ref[...],
                          preferred_element_type=jnp.float32) * d


def _mid_body(p0_ref, p1_ref, hn_ref, d_ref, b_ref, w_ref, o_ref):
    d = d_ref[...]
    t = (p0_ref[...] + p1_ref[...] + hn_ref[...]) * d + b_ref[...]
    t = jnp.maximum(t, 0.0)
    o_ref[...] = jnp.dot(t, w_ref[...], preferred_element_type=jnp.float32) * d


def _fin_make(out_d):
    def _fin_body(p0_ref, p1_ref, hn_ref, d_ref, b_ref, o_ref):
        t = (p0_ref[...] + p1_ref[...] + hn_ref[...]) * d_ref[...] + b_ref[...]
        blk = t.shape[0]
        o_ref[...] = t.reshape(blk, _PK, _F)[:, :, :out_d].reshape(blk * _PK, out_d)
    return _fin_body


def _row_spec(blk):
    return pl.BlockSpec((blk, _PK * _F), lambda i: (i, 0))


def _full_spec(shape):
    return pl.BlockSpec(shape, lambda i: (0, 0))


def _tc_call(body, rows_pk, in_arrays, in_specs, n_out):
    blk = rows_pk // 4
    oshape = jax.ShapeDtypeStruct((rows_pk, _PK * _F), jnp.float32)
    out_shape = [oshape] * n_out if n_out > 1 else oshape
    out_specs = [_row_spec(blk)] * n_out if n_out > 1 else _row_spec(blk)
    return pl.pallas_call(
        body,
        grid=(4,),
        in_specs=in_specs,
        out_specs=out_specs,
        out_shape=out_shape)(*in_arrays)


def kernel(x, edge_index, batch_index, W1, b1, W2, b2, W3, b3):
    f32 = jnp.float32
    n, seq = x.shape
    e = edge_index.shape[1]
    emb = W1.shape[1]
    out_d = W3.shape[1]

    n_pad = _cdiv(n + 1, 1024) * 1024
    rpt_n = n_pad // _NS
    rows_pk = n_pad // _PK
    ct = _cdiv(_cdiv(e, _NS * _K * _LANES), 2 * _UNROLL) * 2 * _UNROLL
    seg_c0 = int(round(ct * 0.72 / _UNROLL)) * _UNROLL
    deg_c0 = int(round(ct * 0.60 / _UNROLL)) * _UNROLL
    rows2d = _NS * _K * ct
    pad = rows2d * _LANES - e

    src2 = jnp.concatenate(
        [edge_index[0], jnp.zeros((pad,), jnp.int32)]).reshape(rows2d, _LANES)
    dst2 = jnp.concatenate(
        [edge_index[1], jnp.full((pad,), n, jnp.int32)]).reshape(rows2d, _LANES)

    eye8 = jnp.eye(_PK, dtype=f32)
    xp = jnp.pad(x, ((0, n_pad - n), (0, _F - seq))).reshape(rows_pk, _PK * _F)
    W1b = jnp.kron(eye8, jnp.pad(W1, ((0, _F - seq), (0, _F - emb))))
    W2b = jnp.kron(eye8, jnp.pad(W2, ((0, _F - emb), (0, _F - emb))))
    W3b = jnp.kron(eye8, jnp.pad(W3, ((0, _F - emb), (0, _F - out_d))))
    b1b = jnp.tile(jnp.pad(b1, (0, _F - emb)), _PK).reshape(1, _PK * _F)
    b2b = jnp.tile(jnp.pad(b2, (0, _F - emb)), _PK).reshape(1, _PK * _F)
    b3b = jnp.tile(jnp.pad(b3, (0, _F - out_d)), _PK).reshape(1, _PK * _F)

    zz = jnp.zeros((n_pad, _F), f32)
    nr = n_pad // 16
    rowidx = jnp.arange(nr, dtype=jnp.int32).reshape(nr // _LANES, _LANES)

    deg_fn = _deg2_make(n_pad, deg_c0, ct - deg_c0)
    seg_fn = _seg_make(n_pad, rpt_n, _F, seg_c0, ct - seg_c0)
    seg_fn1 = seg_fn

    def pk(a):
        return a.reshape(rows_pk, _PK * _F)

    def unpk(a):
        return a.reshape(n_pad, _F)

    dq0, dq1 = deg_fn(dst2.reshape(rows2d * _LANES), zz, rowidx)

    kr = jnp.kron(jnp.eye(_PK, dtype=f32), jnp.ones((1, _F), f32))
    qspec = pl.BlockSpec((rows_pk // 4, _PK), lambda i: (i, 0))
    hn1, dpk = _tc_call(_prep_body, rows_pk,
                        (xp, W1b, kr,
                         dq0.reshape(rows_pk, _PK), dq1.reshape(rows_pk, _PK)),
                        [_row_spec(rows_pk // 4), _full_spec((_PK * _F, _PK * _F)),
                         _full_spec((_PK, _PK * _F)), qspec, qspec], 2)

    s0, s1 = seg_fn1(unpk(hn1), src2, dst2, zz)
    hn2 = _tc_call(_mid_body, rows_pk, (pk(s0), pk(s1), hn1, dpk, b1b, W2b),
                   [_row_spec(rows_pk // 4)] * 4 +
                   [_full_spec((1, _PK * _F)), _full_spec((_PK * _F, _PK * _F))], 1)

    s0, s1 = seg_fn(unpk(hn2), src2, dst2, zz)
    hn3 = _tc_call(_mid_body, rows_pk, (pk(s0), pk(s1), hn2, dpk, b2b, W3b),
                   [_row_spec(rows_pk // 4)] * 4 +
                   [_full_spec((1, _PK * _F)), _full_spec((_PK * _F, _PK * _F))], 1)

    s0, s1 = seg_fn(unpk(hn3), src2, dst2, zz)
    blk = rows_pk // 4
    outp = pl.pallas_call(
        _fin_make(out_d),
        grid=(4,),
        in_specs=[_row_spec(blk)] * 4 + [_full_spec((1, _PK * _F))],
        out_specs=pl.BlockSpec((blk * _PK, out_d), lambda i: (i, 0)),
        out_shape=jax.ShapeDtypeStruct((n, out_d), jnp.float32),
    )(pk(s0), pk(s1), hn3, dpk, b3b)

    return outp


# speedup vs baseline: 1.0478x; 1.0195x over previous
import functools

import jax
import jax.numpy as jnp
from jax import lax
from jax.experimental import pallas as pl
from jax.experimental.pallas import tpu as pltpu
from jax.experimental.pallas import tpu_sc as plsc

_NC = 2
_NS = 16
_LANES = 128
_K = 6
_UNROLL = 6
_F = 16
_PK = 8
_DEGW = 16


def _cdiv(a, b):
    return -(-a // b)


@functools.lru_cache(maxsize=None)
def _seg_make(n_pad, rpt_n, feat, cpt0, cpt1):
    f32 = jnp.float32
    mesh = plsc.VectorSubcoreMesh(core_axis_name="c", subcore_axis_name="s",
                                  num_cores=_NC, num_subcores=_NS)
    rpt_e0, rpt_e1 = cpt0 * _K, cpt1 * _K

    def body(hn, src2, dst2, zz, p0, p1, acc,
             src_a, src_b, dst_a, dst_b, dst_c, rows_a, rows_b,
             lsem, gsem, ssem):
        cid = lax.axis_index("c")
        sid = lax.axis_index("s")
        srcs = (src_a, src_b)
        dsts = (dst_a, dst_b, dst_c)
        rows = (rows_a, rows_b)
        nsl = pl.ds(sid * rpt_n, rpt_n)
        pltpu.sync_copy(zz.at[nsl], acc.at[nsl])
        plsc.subcore_barrier()
        row0 = jnp.where(cid == 0, sid * rpt_e0,
                         _NS * rpt_e0 + sid * rpt_e1)
        cpt = jnp.where(cid == 0, cpt0, cpt1)

        def fire_idx(g, b2, b3):
            base = row0 + g * _K
            pltpu.async_copy(src2.at[pl.ds(base, _K)], srcs[b2], lsem)
            pltpu.async_copy(dst2.at[pl.ds(base, _K)], dsts[b3], lsem)

        def drain_scat(b2):
            for j in range(_K):
                pltpu.make_async_copy(zz.at[pl.ds(0, _LANES)],
                                      rows[b2].at[j], ssem).wait()

        def one_chunk(g, u):
            b2, b3 = u % 2, u % 3

            @pl.when(g >= 2)
            def _():
                drain_scat(b2)
            pltpu.make_async_copy(src2.at[pl.ds(0, _K)], srcs[b2], lsem).wait()
            pltpu.make_async_copy(src2.at[pl.ds(0, _K)], dsts[b3], lsem).wait()

            @pl.when(g + 1 < cpt)
            def _():
                fire_idx(g + 1, (u + 1) % 2, (u + 1) % 3)

            gd = [pltpu.async_copy(hn.at[srcs[b2].at[j]], rows[b2].at[j], gsem)
                  for j in range(_K)]
            for j in range(_K):
                gd[j].wait()
            for j in range(_K):
                pltpu.async_copy(rows[b2].at[j], acc.at[dsts[b3].at[j]],
                                 ssem, add=True)

        fire_idx(0, 0, 0)

        def step(gs, carry):
            for u in range(_UNROLL):
                one_chunk(gs * _UNROLL + u, u)
            return carry

        lax.fori_loop(0, cpt // _UNROLL, step, 0)
        for u in range(2):
            drain_scat(u)
        plsc.subcore_barrier()

        @pl.when(cid == 0)
        def _():
            pltpu.sync_copy(acc.at[nsl], p0.at[nsl])

        @pl.when(cid == 1)
        def _():
            pltpu.sync_copy(acc.at[nsl], p1.at[nsl])

    return pl.kernel(
        body,
        out_type=(jax.ShapeDtypeStruct((n_pad, feat), f32),
                  jax.ShapeDtypeStruct((n_pad, feat), f32)),
        mesh=mesh,
        scratch_types=(pltpu.VMEM_SHARED((n_pad, feat), f32),
                       pltpu.VMEM((_K, _LANES), jnp.int32),
                       pltpu.VMEM((_K, _LANES), jnp.int32),
                       pltpu.VMEM((_K, _LANES), jnp.int32),
                       pltpu.VMEM((_K, _LANES), jnp.int32),
                       pltpu.VMEM((_K, _LANES), jnp.int32),
                       pltpu.VMEM((_K, _LANES, feat), f32),
                       pltpu.VMEM((_K, _LANES, feat), f32),
                       pltpu.SemaphoreType.DMA,
                       pltpu.SemaphoreType.DMA,
                       pltpu.SemaphoreType.DMA),
        compiler_params=pltpu.CompilerParams(use_tc_tiling_on_sc=False))


@functools.lru_cache(maxsize=None)
def _deg2_make(n_pad, cpt0, cpt1):
    f32 = jnp.float32
    i32 = jnp.int32
    mesh = plsc.VectorSubcoreMesh(core_axis_name="c", subcore_axis_name="s",
                                  num_cores=_NC, num_subcores=_NS)
    rpt_e0, rpt_e1 = cpt0 * _K, cpt1 * _K
    nr = n_pad // 16
    nrj = nr // _LANES
    rpt_a = nr // _NS
    epc = _K * _LANES

    def body(dst1d, zz, rowidx, q0, q1, acc, hist, dva, dvb, idx_v,
             lsem, ssem):
        cid = lax.axis_index("c")
        sid = lax.axis_index("s")
        dvs = (dva, dvb)
        nsl = pl.ds(sid * rpt_a, rpt_a)
        pltpu.sync_copy(zz.at[nsl], acc.at[nsl])
        pltpu.sync_copy(zz.at[pl.ds(0, nr)], hist)
        pltpu.sync_copy(rowidx, idx_v)
        plsc.subcore_barrier()
        row0 = jnp.where(cid == 0, sid * rpt_e0,
                         _NS * rpt_e0 + sid * rpt_e1)
        cpt = jnp.where(cid == 0, cpt0, cpt1)
        ones16 = jnp.full((16,), 1.0, f32)

        def fire_idx(g, b2):
            pltpu.async_copy(dst1d.at[pl.ds((row0 + g * _K) * _LANES, epc)],
                             dvs[b2], lsem)

        def one_chunk(g, u):
            b2 = u % 2
            pltpu.make_async_copy(dst1d.at[pl.ds(0, epc)], dvs[b2],
                                  lsem).wait()

            @pl.when(g + 1 < cpt)
            def _():
                fire_idx(g + 1, (u + 1) % 2)

            for r in range(epc // 16):
                iv = dvs[b2][pl.ds(r * 16, 16)]
                plsc.addupdate_scatter(
                    hist, (lax.shift_right_logical(iv, 4),
                           lax.bitwise_and(iv, 15)), ones16)

        fire_idx(0, 0)

        def step(gs, carry):
            for u in range(_UNROLL):
                one_chunk(gs * _UNROLL + u, u)
            return carry

        lax.fori_loop(0, cpt // _UNROLL, step, 0)
        rd = [pltpu.async_copy(hist.at[pl.ds(j * _LANES, _LANES)],
                               acc.at[idx_v.at[j]], ssem, add=True)
              for j in range(nrj)]
        for d in rd:
            d.wait()
        plsc.subcore_barrier()

        @pl.when(cid == 0)
        def _():
            pltpu.sync_copy(acc.at[nsl], q0.at[nsl])

        @pl.when(cid == 1)
        def _():
            pltpu.sync_copy(acc.at[nsl], q1.at[nsl])

    return pl.kernel(
        body,
        out_type=(jax.ShapeDtypeStruct((nr, 16), f32),
                  jax.ShapeDtypeStruct((nr, 16), f32)),
        mesh=mesh,
        scratch_types=(pltpu.VMEM_SHARED((nr, 16), f32),
                       pltpu.VMEM((nr, 16), f32),
                       pltpu.VMEM((epc,), i32),
                       pltpu.VMEM((epc,), i32),
                       pltpu.VMEM((nrj, _LANES), i32),
                       pltpu.SemaphoreType.DMA,
                       pltpu.SemaphoreType.DMA),
        compiler_params=pltpu.CompilerParams(use_tc_tiling_on_sc=False,
                                             needs_layout_passes=False))


@functools.lru_cache(maxsize=None)
def _deg_make(n_pad, rpt_n, feat, cpt0, cpt1):
    f32 = jnp.float32
    mesh = plsc.VectorSubcoreMesh(core_axis_name="c", subcore_axis_name="s",
                                  num_cores=_NC, num_subcores=_NS)
    rpt_e0, rpt_e1 = cpt0 * _K, cpt1 * _K

    def body(dst2, zz, ones, q0, q1, accd, dst_a, dst_b, dst_c, ones_v,
             lsem, ssem):
        cid = lax.axis_index("c")
        sid = lax.axis_index("s")
        dsts = (dst_a, dst_b, dst_c)
        nsl = pl.ds(sid * rpt_n, rpt_n)
        pltpu.sync_copy(zz.at[nsl], accd.at[nsl])
        pltpu.sync_copy(ones, ones_v)
        plsc.subcore_barrier()
        row0 = jnp.where(cid == 0, sid * rpt_e0,
                         _NS * rpt_e0 + sid * rpt_e1)
        cpt = jnp.where(cid == 0, cpt0, cpt1)

        def fire_idx(g, b3):
            base = row0 + g * _K
            pltpu.async_copy(dst2.at[pl.ds(base, _K)], dsts[b3], lsem)

        def drain_scat():
            for j in range(_K):
                pltpu.make_async_copy(zz.at[pl.ds(0, _LANES)],
                                      ones_v, ssem).wait()

        def one_chunk(g, u):
            b3 = u % 3

            @pl.when(g >= 2)
            def _():
                drain_scat()

            pltpu.make_async_copy(dst2.at[pl.ds(0, _K)], dsts[b3], lsem).wait()

            @pl.when(g + 1 < cpt)
            def _():
                fire_idx(g + 1, (u + 1) % 3)

            for j in range(_K):
                pltpu.async_copy(ones_v, accd.at[dsts[b3].at[j]],
                                 ssem, add=True)

        fire_idx(0, 0)

        def step(gs, carry):
            for u in range(_UNROLL):
                one_chunk(gs * _UNROLL + u, u)
            return carry

        lax.fori_loop(0, cpt // _UNROLL, step, 0)
        for _u in range(2):
            drain_scat()
        plsc.subcore_barrier()

        @pl.when(cid == 0)
        def _():
            pltpu.sync_copy(accd.at[nsl], q0.at[nsl])

        @pl.when(cid == 1)
        def _():
            pltpu.sync_copy(accd.at[nsl], q1.at[nsl])

    return pl.kernel(
        body,
        out_type=(jax.ShapeDtypeStruct((n_pad, feat), f32),
                  jax.ShapeDtypeStruct((n_pad, feat), f32)),
        mesh=mesh,
        scratch_types=(pltpu.VMEM_SHARED((n_pad, feat), f32),
                       pltpu.VMEM((_K, _LANES), jnp.int32),
                       pltpu.VMEM((_K, _LANES), jnp.int32),
                       pltpu.VMEM((_K, _LANES), jnp.int32),
                       pltpu.VMEM((_LANES, feat), f32),
                       pltpu.SemaphoreType.DMA,
                       pltpu.SemaphoreType.DMA),
        compiler_params=pltpu.CompilerParams(use_tc_tiling_on_sc=False))


def _prep_body(x_ref, w_ref, kr_ref, q0_ref, q1_ref, hn_ref, d_ref):
    dn = lax.rsqrt(q0_ref[...] + q1_ref[...] + 1.0)
    d = jnp.dot(dn, kr_ref[...], preferred_element_type=jnp.float32)
    d_ref[...] = d
    hn_ref[...] = jnp.dot(x_ref[...], w_ref[...],
                          preferred_element_type=jnp.float32) * d


def _mid_body(p0_ref, p1_ref, hn_ref, d_ref, b_ref, w_ref, o_ref):
    d = d_ref[...]
    t = (p0_ref[...] + p1_ref[...] + hn_ref[...]) * d + b_ref[...]
    t = jnp.maximum(t, 0.0)
    o_ref[...] = jnp.dot(t, w_ref[...], preferred_element_type=jnp.float32) * d


def _fin_make(out_d):
    def _fin_body(p0_ref, p1_ref, hn_ref, d_ref, b_ref, o_ref):
        t = (p0_ref[...] + p1_ref[...] + hn_ref[...]) * d_ref[...] + b_ref[...]
        blk = t.shape[0]
        o_ref[...] = t.reshape(blk, _PK, _F)[:, :, :out_d].reshape(blk * _PK, out_d)
    return _fin_body


def _row_spec(blk):
    return pl.BlockSpec((blk, _PK * _F), lambda i: (i, 0))


def _full_spec(shape):
    return pl.BlockSpec(shape, lambda i: (0, 0))


def _tc_call(body, rows_pk, in_arrays, in_specs, n_out):
    blk = rows_pk // 4
    oshape = jax.ShapeDtypeStruct((rows_pk, _PK * _F), jnp.float32)
    out_shape = [oshape] * n_out if n_out > 1 else oshape
    out_specs = [_row_spec(blk)] * n_out if n_out > 1 else _row_spec(blk)
    return pl.pallas_call(
        body,
        grid=(4,),
        in_specs=in_specs,
        out_specs=out_specs,
        out_shape=out_shape)(*in_arrays)


def kernel(x, edge_index, batch_index, W1, b1, W2, b2, W3, b3):
    f32 = jnp.float32
    n, seq = x.shape
    e = edge_index.shape[1]
    emb = W1.shape[1]
    out_d = W3.shape[1]

    n_pad = _cdiv(n + 1, 1024) * 1024
    rpt_n = n_pad // _NS
    rows_pk = n_pad // _PK
    ct = _cdiv(_cdiv(e, _NS * _K * _LANES), 2 * _UNROLL) * 2 * _UNROLL
    seg_c0 = int(round(ct * 0.76 / _UNROLL)) * _UNROLL
    deg_c0 = int(round(ct * 0.60 / _UNROLL)) * _UNROLL
    rows2d = _NS * _K * ct
    pad = rows2d * _LANES - e

    src2 = jnp.concatenate(
        [edge_index[0], jnp.zeros((pad,), jnp.int32)]).reshape(rows2d, _LANES)
    dst2 = jnp.concatenate(
        [edge_index[1], jnp.full((pad,), n, jnp.int32)]).reshape(rows2d, _LANES)

    eye8 = jnp.eye(_PK, dtype=f32)
    xp = jnp.pad(x, ((0, n_pad - n), (0, _F - seq))).reshape(rows_pk, _PK * _F)
    W1b = jnp.kron(eye8, jnp.pad(W1, ((0, _F - seq), (0, _F - emb))))
    W2b = jnp.kron(eye8, jnp.pad(W2, ((0, _F - emb), (0, _F - emb))))
    W3b = jnp.kron(eye8, jnp.pad(W3, ((0, _F - emb), (0, _F - out_d))))
    b1b = jnp.tile(jnp.pad(b1, (0, _F - emb)), _PK).reshape(1, _PK * _F)
    b2b = jnp.tile(jnp.pad(b2, (0, _F - emb)), _PK).reshape(1, _PK * _F)
    b3b = jnp.tile(jnp.pad(b3, (0, _F - out_d)), _PK).reshape(1, _PK * _F)

    zz = jnp.zeros((n_pad, _F), f32)
    nr = n_pad // 16
    rowidx = jnp.arange(nr, dtype=jnp.int32).reshape(nr // _LANES, _LANES)

    deg_fn = _deg2_make(n_pad, deg_c0, ct - deg_c0)
    seg_fn = _seg_make(n_pad, rpt_n, _F, seg_c0, ct - seg_c0)
    seg_fn1 = seg_fn

    def pk(a):
        return a.reshape(rows_pk, _PK * _F)

    def unpk(a):
        return a.reshape(n_pad, _F)

    dq0, dq1 = deg_fn(dst2.reshape(rows2d * _LANES), zz, rowidx)

    kr = jnp.kron(jnp.eye(_PK, dtype=f32), jnp.ones((1, _F), f32))
    qspec = pl.BlockSpec((rows_pk // 4, _PK), lambda i: (i, 0))
    hn1, dpk = _tc_call(_prep_body, rows_pk,
                        (xp, W1b, kr,
                         dq0.reshape(rows_pk, _PK), dq1.reshape(rows_pk, _PK)),
                        [_row_spec(rows_pk // 4), _full_spec((_PK * _F, _PK * _F)),
                         _full_spec((_PK, _PK * _F)), qspec, qspec], 2)

    s0, s1 = seg_fn1(unpk(hn1), src2, dst2, zz)
    hn2 = _tc_call(_mid_body, rows_pk, (pk(s0), pk(s1), hn1, dpk, b1b, W2b),
                   [_row_spec(rows_pk // 4)] * 4 +
                   [_full_spec((1, _PK * _F)), _full_spec((_PK * _F, _PK * _F))], 1)

    s0, s1 = seg_fn(unpk(hn2), src2, dst2, zz)
    hn3 = _tc_call(_mid_body, rows_pk, (pk(s0), pk(s1), hn2, dpk, b2b, W3b),
                   [_row_spec(rows_pk // 4)] * 4 +
                   [_full_spec((1, _PK * _F)), _full_spec((_PK * _F, _PK * _F))], 1)

    s0, s1 = seg_fn(unpk(hn3), src2, dst2, zz)
    blk = rows_pk // 4
    outp = pl.pallas_call(
        _fin_make(out_d),
        grid=(4,),
        in_specs=[_row_spec(blk)] * 4 + [_full_spec((1, _PK * _F))],
        out_specs=pl.BlockSpec((blk * _PK, out_d), lambda i: (i, 0)),
        out_shape=jax.ShapeDtypeStruct((n, out_d), jnp.float32),
    )(pk(s0), pk(s1), hn3, dpk, b3b)

    return outp
